# baseline (speedup 1.0000x reference)
import functools

import jax
import jax.numpy as jnp
from jax import lax
from jax.experimental import pallas as pl
from jax.experimental.pallas import tpu as pltpu
from jax.experimental.pallas import tpu_sc as plsc

N_FACTORS = 32
BATCH = 16384
NUM_CORES = 2
NUM_SUBCORES = 16
NUM_WORKERS = NUM_CORES * NUM_SUBCORES
PAIRS_PER_WORKER = BATCH // NUM_WORKERS
CHUNK = 128
NUM_CHUNKS = PAIRS_PER_WORKER // CHUNK
LANES = 16
NUM_GROUPS = PAIRS_PER_WORKER // LANES


def _body(data_hbm, uf_hbm, if_hbm, out_hbm,
          data_v, uidx_v, iidx_v, urows_v, irows_v, out_v, sem):
    wid = lax.axis_index("s") * NUM_CORES + lax.axis_index("c")
    base = wid * PAIRS_PER_WORKER

    pltpu.sync_copy(data_hbm.at[pl.ds(base, PAIRS_PER_WORKER), :], data_v)

    lane = lax.iota(jnp.int32, 16)
    zeros = jnp.zeros((16,), jnp.int32)
    ones = jnp.ones((16,), jnp.int32)
    for g in range(NUM_GROUPS):
        rows = g * LANES + lane
        u_ids = plsc.load_gather(data_v, [rows, zeros])
        i_ids = plsc.load_gather(data_v, [rows, ones])
        c, o = divmod(g * LANES, CHUNK)
        uidx_v[c, pl.ds(o, LANES)] = u_ids
        iidx_v[c, pl.ds(o, LANES)] = i_ids

    copies = []
    for k in range(NUM_CHUNKS):
        copies.append(pltpu.async_copy(
            uf_hbm.at[uidx_v.at[k]], urows_v.at[k], sem))
        copies.append(pltpu.async_copy(
            if_hbm.at[iidx_v.at[k]], irows_v.at[k], sem))
    for c in copies:
        c.wait()

    def group_body(g, carry):
        rows = g * LANES + lane
        chunk = rows // CHUNK
        row_in = rows % CHUNK
        acc = jnp.zeros((16,), jnp.float32)
        for j in range(N_FACTORS):
            cj = jnp.full((16,), j, jnp.int32)
            uu = plsc.load_gather(urows_v, [chunk, row_in, cj])
            vv = plsc.load_gather(irows_v, [chunk, row_in, cj])
            acc = acc + uu * vv
        out_v[pl.ds(g * LANES, LANES)] = acc
        return carry

    lax.fori_loop(0, NUM_GROUPS, group_body, 0)

    pltpu.sync_copy(out_v, out_hbm.at[pl.ds(base, PAIRS_PER_WORKER)])


@jax.jit
def kernel(data, user_factors, item_factors):
    mesh = plsc.VectorSubcoreMesh(
        core_axis_name="c", subcore_axis_name="s",
        num_cores=NUM_CORES, num_subcores=NUM_SUBCORES)
    run = pl.kernel(
        _body,
        jax.ShapeDtypeStruct((BATCH,), jnp.float32),
        mesh=mesh,
        compiler_params=pltpu.CompilerParams(
            needs_layout_passes=False, use_tc_tiling_on_sc=False),
        scratch_types=[
            pltpu.VMEM((PAIRS_PER_WORKER, 2), jnp.int32),
            pltpu.VMEM((NUM_CHUNKS, CHUNK), jnp.int32),
            pltpu.VMEM((NUM_CHUNKS, CHUNK), jnp.int32),
            pltpu.VMEM((NUM_CHUNKS, CHUNK, N_FACTORS), jnp.float32),
            pltpu.VMEM((NUM_CHUNKS, CHUNK, N_FACTORS), jnp.float32),
            pltpu.VMEM((PAIRS_PER_WORKER,), jnp.float32),
            pltpu.SemaphoreType.DMA,
        ],
    )
    return run(data.astype(jnp.int32), user_factors, item_factors)
